# baseline (speedup 1.0000x reference)
import jax
import jax.numpy as jnp
from jax import lax
from jax.experimental import pallas as pl
from jax.experimental.pallas import tpu as pltpu
from jax.experimental.pallas import tpu_sc as plsc

_ROWS = 4096
_LK = 4096
_NC, _NS = 2, 16
_NW = _NC * _NS
_RPW = _ROWS // _NW
_CPR = 8
_CPW = _RPW * _CPR
_GROUP = 128
_NGRP = _CPW // _GROUP
_CB = _LK // _CPR


def _sc_body(mt_hbm, table_hbm, out_hbm, mt_v, idx_v, buf_v, sem):
    c = lax.axis_index("c")
    s = lax.axis_index("s")
    wid = s * _NC + c
    row0 = wid * _RPW
    pltpu.sync_copy(mt_hbm.at[pl.ds(row0, _RPW)], mt_v)
    lane = lax.iota(jnp.int32, 16)
    for i in range(_CPW // 16):
        g = lane + 16 * i
        r = lax.shift_right_logical(g, 3)
        ch = lax.bitwise_and(g, 7)
        m = plsc.load_gather(mt_v, [r])
        cm = lax.shift_right_logical(m, 9)
        p = lax.bitwise_and(m, 511)
        idx = jnp.where(ch < cm, 511, jnp.where(ch == cm, p, 512))
        idx_v[i // 8, pl.ds((i % 8) * 16, 16)] = idx
    for j in range(_NGRP):
        pltpu.async_copy(table_hbm.at[idx_v.at[j]], buf_v, sem).wait()
        pltpu.sync_copy(buf_v, out_hbm.at[pl.ds(wid * _CPW + j * _GROUP, _GROUP)])


def kernel(m_top, scores):
    B, H, U, L_K = scores.shape
    mt = m_top.reshape(_ROWS).astype(jnp.int32)
    pcol = jnp.arange(_CB, dtype=jnp.int32)
    prow = jnp.arange(_CB + 1, dtype=jnp.int32)
    table = (pcol[None, :] > prow[:, None]) | (prow[:, None] == _CB)

    k = pl.kernel(
        _sc_body,
        out_type=jax.ShapeDtypeStruct((_ROWS * _CPR, _CB), jnp.bool_),
        compiler_params=pltpu.CompilerParams(needs_layout_passes=False),
        mesh=plsc.VectorSubcoreMesh(
            core_axis_name="c", subcore_axis_name="s",
            num_cores=_NC, num_subcores=_NS,
        ),
        scratch_types=[
            pltpu.VMEM((_RPW,), jnp.int32),
            pltpu.VMEM((_NGRP, _GROUP), jnp.int32),
            pltpu.VMEM((_GROUP, _CB), jnp.bool_),
            pltpu.SemaphoreType.DMA,
        ],
    )
    out = k(mt, table)
    return out.reshape(B, H, U, L_K)


# speedup vs baseline: 6.2877x; 6.2877x over previous
import jax
import jax.numpy as jnp
from jax import lax
from jax.experimental import pallas as pl
from jax.experimental.pallas import tpu as pltpu
from jax.experimental.pallas import tpu_sc as plsc

_ROWS = 4096
_LK = 4096
_NC, _NS = 2, 16
_NW = _NC * _NS
_RPW = _ROWS // _NW
_CPR = 8
_CPW = _RPW * _CPR
_GROUP = 128
_NGRP = _CPW // _GROUP
_CB = _LK // _CPR


def _sc_body(mt_hbm, table_hbm, out_hbm, mt_v, idx_v, buf_v, sem):
    c = lax.axis_index("c")
    s = lax.axis_index("s")
    wid = s * _NC + c
    row0 = wid * _RPW
    pltpu.sync_copy(mt_hbm.at[pl.ds(row0, _RPW)], mt_v)
    lane = lax.iota(jnp.int32, 16)
    for i in range(_CPW // 16):
        g = lane + 16 * i
        r = lax.shift_right_logical(g, 3)
        ch = lax.bitwise_and(g, 7)
        m = plsc.load_gather(mt_v, [r])
        cm = lax.shift_right_logical(m, 9)
        p = lax.bitwise_and(m, 511)
        idx = jnp.where(ch < cm, 511, jnp.where(ch == cm, p, 512))
        idx_v[i // 8, pl.ds((i % 8) * 16, 16)] = idx
    for j in range(_NGRP):
        pltpu.sync_copy(buf_v, out_hbm.at[pl.ds(wid * _CPW + j * _GROUP, _GROUP)])


def kernel(m_top, scores):
    B, H, U, L_K = scores.shape
    mt = m_top.reshape(_ROWS).astype(jnp.int32)
    pcol = jnp.arange(_CB, dtype=jnp.int32)
    prow = jnp.arange(_CB + 1, dtype=jnp.int32)
    table = (pcol[None, :] > prow[:, None]) | (prow[:, None] == _CB)

    k = pl.kernel(
        _sc_body,
        out_type=jax.ShapeDtypeStruct((_ROWS * _CPR, _CB), jnp.bool_),
        compiler_params=pltpu.CompilerParams(needs_layout_passes=False),
        mesh=plsc.VectorSubcoreMesh(
            core_axis_name="c", subcore_axis_name="s",
            num_cores=_NC, num_subcores=_NS,
        ),
        scratch_types=[
            pltpu.VMEM((_RPW,), jnp.int32),
            pltpu.VMEM((_NGRP, _GROUP), jnp.int32),
            pltpu.VMEM((_GROUP, _CB), jnp.bool_),
            pltpu.SemaphoreType.DMA,
        ],
    )
    out = k(mt, table)
    return out.reshape(B, H, U, L_K)


# speedup vs baseline: 49.4371x; 7.8625x over previous
import jax
import jax.numpy as jnp
from jax.experimental import pallas as pl

_BLK_ROWS = 512


def _mask_kernel(mtop_ref, out_ref):
    mtop = mtop_ref[...]
    cols = jax.lax.broadcasted_iota(jnp.int32, out_ref.shape, 1)
    out_ref[...] = (cols > mtop).astype(jnp.int8)


def kernel(m_top, scores):
    B, H, U, L_K = scores.shape
    rows = B * H * U
    grid = rows // _BLK_ROWS
    mt = m_top.reshape(rows, 1).astype(jnp.int32)
    out = pl.pallas_call(
        _mask_kernel,
        grid=(grid,),
        in_specs=[pl.BlockSpec((_BLK_ROWS, 1), lambda i: (i, 0))],
        out_specs=pl.BlockSpec((_BLK_ROWS, L_K), lambda i: (i, 0)),
        out_shape=jax.ShapeDtypeStruct((rows, L_K), jnp.int8),
    )(mt)
    return out.reshape(B, H, U, L_K)
